# baseline (speedup 1.0000x reference)
import functools

import numpy as np
import jax
import jax.numpy as jnp
from jax import lax
from jax.experimental import pallas as pl
from jax.experimental.pallas import tpu as pltpu
from jax.experimental.pallas import tpu_sc as plsc

_B = 4
_N1 = 1024
_M1 = 256
_M2 = 64
_K = 128
_R2_1 = 0.4 * 0.4
_R2_2 = 0.8 * 0.8
_NEG = -3.0e38


def _fps_kernel(px_ref, py_ref, q1x_ref, q1y_ref, q2x_ref, q2y_ref):
    def run(px, py, m, qx_ref, qy_ref):
        b, n = px.shape
        iota = lax.broadcasted_iota(jnp.int32, (1, n), 1)
        miota = lax.broadcasted_iota(jnp.int32, (1, m), 1)

        def body(i, carry):
            dist, last, qx, qy = carry
            sel = (iota == last).astype(jnp.float32)
            lx = jnp.sum(px * sel, axis=1, keepdims=True)
            ly = jnp.sum(py * sel, axis=1, keepdims=True)
            at = miota == (i - 1)
            qx = jnp.where(at, lx, qx)
            qy = jnp.where(at, ly, qy)
            d = (px - lx) ** 2 + (py - ly) ** 2
            dist = jnp.minimum(dist, d)
            mx = jnp.max(dist, axis=1, keepdims=True)
            nxt = jnp.min(jnp.where(dist == mx, iota, n), axis=1, keepdims=True)
            return dist, nxt, qx, qy

        dist0 = jnp.full((b, n), jnp.inf, dtype=jnp.float32)
        last0 = jnp.zeros((b, 1), dtype=jnp.int32)
        qx0 = jnp.zeros((b, m), dtype=jnp.float32)
        _, _, qx, qy = lax.fori_loop(1, m + 1, body, (dist0, last0, qx0, qx0))
        qx_ref[...] = qx
        qy_ref[...] = qy

    run(px_ref[...], py_ref[...], _M1, q1x_ref, q1y_ref)
    run(q1x_ref[...], q1y_ref[...], _M2, q2x_ref, q2y_ref)


def _fps(px, py):
    f32 = jnp.float32
    return pl.pallas_call(
        _fps_kernel,
        out_shape=[
            jax.ShapeDtypeStruct((_B, _M1), f32),
            jax.ShapeDtypeStruct((_B, _M1), f32),
            jax.ShapeDtypeStruct((_B, _M2), f32),
            jax.ShapeDtypeStruct((_B, _M2), f32),
        ],
    )(px, py)


def _atab_kernel(x_ref, sx_ref, sy_ref, w_ref, b_ref, wpx_ref, wpy_ref, o_ref):
    x = x_ref[...].reshape(x_ref.shape[1:])
    sx = sx_ref[...].reshape(sx_ref.shape[1:])
    sy = sy_ref[...].reshape(sy_ref.shape[1:])
    a = jnp.dot(x, w_ref[...], preferred_element_type=jnp.float32)
    a = a + b_ref[...] + sx * wpx_ref[...] + sy * wpy_ref[...]
    o_ref[...] = a.reshape(o_ref.shape)


def _atab(x, sx3, sy3, w, b, wpx, wpy):
    bsz, n, c = x.shape
    h = w.shape[1]
    return pl.pallas_call(
        _atab_kernel,
        grid=(bsz,),
        in_specs=[
            pl.BlockSpec((1, n, c), lambda i: (i, 0, 0)),
            pl.BlockSpec((1, n, 1), lambda i: (i, 0, 0)),
            pl.BlockSpec((1, n, 1), lambda i: (i, 0, 0)),
            pl.BlockSpec((c, h), lambda i: (0, 0)),
            pl.BlockSpec((1, h), lambda i: (0, 0)),
            pl.BlockSpec((1, h), lambda i: (0, 0)),
            pl.BlockSpec((1, h), lambda i: (0, 0)),
        ],
        out_specs=pl.BlockSpec((1, n, h), lambda i: (i, 0, 0)),
        out_shape=jax.ShapeDtypeStruct((bsz, n, h), jnp.float32),
    )(x, sx3, sy3, w, b, wpx, wpy)


def _bq_kernel(qx_ref, qy_ref, wpx_ref, wpy_ref, o_ref):
    qx = qx_ref[...].reshape(qx_ref.shape[1:])
    qy = qy_ref[...].reshape(qy_ref.shape[1:])
    o = -(qx * wpx_ref[...] + qy * wpy_ref[...])
    o_ref[...] = o.reshape(o_ref.shape)


def _bq(qx3, qy3, wpx, wpy):
    bsz, m, _ = qx3.shape
    h = wpx.shape[1]
    return pl.pallas_call(
        _bq_kernel,
        grid=(bsz,),
        in_specs=[
            pl.BlockSpec((1, m, 1), lambda i: (i, 0, 0)),
            pl.BlockSpec((1, m, 1), lambda i: (i, 0, 0)),
            pl.BlockSpec((1, h), lambda i: (0, 0)),
            pl.BlockSpec((1, h), lambda i: (0, 0)),
        ],
        out_specs=pl.BlockSpec((1, m, h), lambda i: (i, 0, 0)),
        out_shape=jax.ShapeDtypeStruct((bsz, m, h), jnp.float32),
    )(qx3, qy3, wpx, wpy)


_BIGSLOT = 1 << 29


def _nbr_kernel(sx_ref, sy_ref, qx_ref, qy_ref, ltri_ref, slot_ref, cnt_ref,
                *, r2):
    sx = sx_ref[...].reshape(sx_ref.shape[2:])
    sy = sy_ref[...].reshape(sy_ref.shape[2:])
    qx = qx_ref[...].reshape(qx_ref.shape[1:])
    qy = qy_ref[...].reshape(qy_ref.shape[1:])
    dx = sx - qx
    dy = sy - qy
    m = ((dx * dx + dy * dy) <= r2).astype(jnp.float32)
    prefix = jnp.dot(m, ltri_ref[...], preferred_element_type=jnp.float32)
    keep = (m > 0.5) & (prefix <= float(_K))
    slot = jnp.where(keep, prefix - 1.0, float(_BIGSLOT)).astype(jnp.int32)
    slot_ref[...] = slot.reshape(slot_ref.shape)
    cnt = jnp.minimum(jnp.sum(m, axis=1, keepdims=True), float(_K))
    cnt_ref[...] = cnt.astype(jnp.int32).reshape(cnt_ref.shape)


def _nbr(sx4, sy4, qx3, qy3, ltri, mq, tq, r2):
    bsz = qx3.shape[0]
    n = ltri.shape[0]
    nq = mq // tq
    kern = functools.partial(_nbr_kernel, r2=r2)
    return pl.pallas_call(
        kern,
        grid=(bsz, nq),
        in_specs=[
            pl.BlockSpec((1, 1, 1, n), lambda b, q: (b, 0, 0, 0)),
            pl.BlockSpec((1, 1, 1, n), lambda b, q: (b, 0, 0, 0)),
            pl.BlockSpec((1, tq, 1), lambda b, q: (b, q, 0)),
            pl.BlockSpec((1, tq, 1), lambda b, q: (b, q, 0)),
            pl.BlockSpec((n, n), lambda b, q: (0, 0)),
        ],
        out_specs=[
            pl.BlockSpec((1, tq, n), lambda b, q: (b, q, 0)),
            pl.BlockSpec((1, tq, 1), lambda b, q: (b, q, 0)),
        ],
        out_shape=[
            jax.ShapeDtypeStruct((bsz, mq, n), jnp.int32),
            jax.ShapeDtypeStruct((bsz, mq, 1), jnp.int32),
        ],
        compiler_params=pltpu.CompilerParams(
            dimension_semantics=("parallel", "parallel"),
        ),
    )(sx4, sy4, qx3, qy3, ltri)


def _sc_gather(a_tab, slot_f, cnt_f, *, n, mq, h):
    info = plsc.get_sparse_core_info()
    nw = info.num_cores * info.num_subcores
    nq_total = _B * mq
    qw = max(nq_total // nw, 16)
    active = nq_total // qw
    nblk = qw // 16
    mesh = plsc.VectorSubcoreMesh(core_axis_name="c", subcore_axis_name="s")
    f32 = jnp.float32

    @functools.partial(
        pl.kernel,
        mesh=mesh,
        out_type=[
            jax.ShapeDtypeStruct((nq_total * _K, h), f32),
            jax.ShapeDtypeStruct((nq_total * _K,), f32),
        ],
        scratch_types=[
            pltpu.VMEM((n,), jnp.int32),
            pltpu.VMEM((16,), jnp.int32),
            pltpu.VMEM((_K + 16,), jnp.int32),
            pltpu.VMEM((_K,), jnp.int32),
            pltpu.VMEM((_K,), f32),
            pltpu.VMEM((_K, h), f32),
            pltpu.SemaphoreType.DMA,
        ],
        compiler_params=pltpu.CompilerParams(needs_layout_passes=False),
    )
    def k(a_hbm, slot_hbm, cnt_hbm, ag_hbm, pen_hbm,
          slotv, cntv, idxbig, idxg, penv, rows, sem):
        wid = lax.axis_index("s") * info.num_cores + lax.axis_index("c")

        @pl.when(wid < active)
        def _work():
            q0 = wid * qw
            cloud = q0 // mq
            base = cloud * n
            liota = lax.iota(jnp.int32, 16)

            def clr(kk, _):
                idxbig[pl.ds(kk * 16, 16)] = jnp.zeros((16,), jnp.int32)
                return 0
            lax.fori_loop(0, (_K + 16) // 16, clr, 0)

            def per_query(cnt, qid):
                pltpu.sync_copy(slot_hbm.at[pl.ds(qid * n, n)], slotv)

                def scan(j, _):
                    sv = slotv[pl.ds(j * 16, 16)]
                    sidx = liota + (j * 16 + base)
                    dst = jnp.where(sv < _K, sv, _K + liota)
                    plsc.store_scatter(idxbig, [dst], sidx)
                    return 0
                lax.fori_loop(0, n // 16, scan, 0)

                def fin(kk, _):
                    sl = pl.ds(kk * 16, 16)
                    idxg[sl] = idxbig[sl]
                    penv[sl] = jnp.where(liota + kk * 16 < cnt, 0.0, _NEG)
                    return 0
                lax.fori_loop(0, _K // 16, fin, 0)

                pltpu.async_copy(a_hbm.at[idxg], rows, sem).wait()
                pltpu.sync_copy(rows, ag_hbm.at[pl.ds(qid * _K, _K)])
                pltpu.sync_copy(penv, pen_hbm.at[pl.ds(qid * _K, _K)])

            def blk(g, _):
                pltpu.sync_copy(cnt_hbm.at[pl.ds(q0 + g * 16, 16)], cntv)
                cv = cntv[...]
                for l in range(16):
                    per_query(cv[l], q0 + g * 16 + l)
                return 0
            lax.fori_loop(0, nblk, blk, 0)

    ag, pen = k(a_tab, slot_f, cnt_f)
    return ag, pen


def _sa2_kernel(ag_ref, bq_ref, pen_ref, wb_ref, bb_ref, o_ref):
    tqk, h = ag_ref.shape[1], ag_ref.shape[2]
    tq = tqk // _K
    ag = ag_ref[...].reshape(tqk, h)
    bq = bq_ref[...].reshape(tq, 1, h)
    pre = ag + jnp.broadcast_to(bq, (tq, _K, h)).reshape(tqk, h)
    t = jnp.tanh(pre)
    ho = wb_ref.shape[1]
    hh = jnp.dot(t, wb_ref[...], preferred_element_type=jnp.float32)
    hh = hh + bb_ref[...]
    pen = pen_ref[...].reshape(tq, _K)
    h3 = hh.reshape(tq, _K, ho) + pen[:, :, None]
    o_ref[...] = jnp.max(h3, axis=1).reshape(o_ref.shape)


def _sa2(ag, bq5, pen, wb, bb, m, tq):
    bsz = ag.shape[0]
    h = ag.shape[2]
    ho = wb.shape[1]
    nq = m // tq
    return pl.pallas_call(
        _sa2_kernel,
        grid=(bsz, nq),
        in_specs=[
            pl.BlockSpec((1, tq * _K, h), lambda b, q: (b, q, 0)),
            pl.BlockSpec((1, 1, tq, 1, h), lambda b, q: (b, q, 0, 0, 0)),
            pl.BlockSpec((1, tq, _K), lambda b, q: (b, q, 0)),
            pl.BlockSpec((h, ho), lambda b, q: (0, 0)),
            pl.BlockSpec((1, ho), lambda b, q: (0, 0)),
        ],
        out_specs=pl.BlockSpec((1, tq, ho), lambda b, q: (b, q, 0)),
        out_shape=jax.ShapeDtypeStruct((bsz, m, ho), jnp.float32),
        compiler_params=pltpu.CompilerParams(
            dimension_semantics=("parallel", "parallel"),
        ),
    )(ag, bq5, pen, wb, bb)


def _global_kernel(x2_ref, qx_ref, qy_ref, w3x_ref, w3px_ref, w3py_ref,
                   b3a_ref, w3b_ref, b3b_ref, o_ref):
    x2 = x2_ref[...].reshape(x2_ref.shape[1:])
    qx = qx_ref[...].reshape(qx_ref.shape[1:])
    qy = qy_ref[...].reshape(qy_ref.shape[1:])
    pre = jnp.dot(x2, w3x_ref[...], preferred_element_type=jnp.float32)
    pre = pre + qx * w3px_ref[...] + qy * w3py_ref[...] + b3a_ref[...]
    hh = jnp.dot(jnp.tanh(pre), w3b_ref[...],
                 preferred_element_type=jnp.float32) + b3b_ref[...]
    o_ref[...] = jnp.max(hh, axis=0, keepdims=True)[None]


def _global(x2, qx3, qy3, w3x, w3px, w3py, b3a, w3b, b3b):
    bsz, m, c = x2.shape
    h1 = w3x.shape[1]
    h2 = w3b.shape[1]
    return pl.pallas_call(
        _global_kernel,
        grid=(bsz,),
        in_specs=[
            pl.BlockSpec((1, m, c), lambda i: (i, 0, 0)),
            pl.BlockSpec((1, m, 1), lambda i: (i, 0, 0)),
            pl.BlockSpec((1, m, 1), lambda i: (i, 0, 0)),
            pl.BlockSpec((c, h1), lambda i: (0, 0)),
            pl.BlockSpec((1, h1), lambda i: (0, 0)),
            pl.BlockSpec((1, h1), lambda i: (0, 0)),
            pl.BlockSpec((1, h1), lambda i: (0, 0)),
            pl.BlockSpec((h1, h2), lambda i: (0, 0)),
            pl.BlockSpec((1, h2), lambda i: (0, 0)),
        ],
        out_specs=pl.BlockSpec((1, 1, h2), lambda i: (i, 0, 0)),
        out_shape=jax.ShapeDtypeStruct((bsz, 1, h2), jnp.float32),
    )(x2, qx3, qy3, w3x, w3px, w3py, b3a, w3b, b3b)


def _stage(x_src, sx, sy, qx, qy, wa, ba, wb, bb, *, n, mq, r2, tq, nbr_tq):
    c = x_src.shape[2]
    wx, wpx, wpy = wa[:c], wa[c:c + 1], wa[c + 1:c + 2]
    a = _atab(x_src, sx[:, :, None], sy[:, :, None], wx, ba[None], wpx, wpy)
    bq = _bq(qx[:, :, None], qy[:, :, None], wpx, wpy)
    h = a.shape[2]
    ltri = jnp.tril(jnp.ones((n, n), dtype=jnp.float32))
    slot, cnt = _nbr(sx.reshape(_B, 1, 1, n), sy.reshape(_B, 1, 1, n),
                     qx[:, :, None], qy[:, :, None], ltri, mq, nbr_tq, r2)
    ag, pen = _sc_gather(a.reshape(_B * n, h),
                         slot.reshape(-1), cnt.reshape(-1),
                         n=n, mq=mq, h=h)
    nq = mq // tq
    out = _sa2(ag.reshape(_B, mq * _K, h),
               bq.reshape(_B, nq, tq, 1, h),
               pen.reshape(_B, mq, _K),
               wb, bb[None], mq, tq)
    return out


def kernel(x, pos, W1a, b1a, W1b, b1b, W2a, b2a, W2b, b2b, W3a, b3a, W3b, b3b):
    px = pos[:, :, 0]
    py = pos[:, :, 1]
    q1x, q1y, q2x, q2y = _fps(px, py)

    x1 = _stage(x, px, py, q1x, q1y, W1a, b1a, W1b, b1b,
                n=_N1, mq=_M1, r2=_R2_1, tq=64, nbr_tq=64)
    x2 = _stage(x1, q1x, q1y, q2x, q2y, W2a, b2a, W2b, b2b,
                n=_M1, mq=_M2, r2=_R2_2, tq=32, nbr_tq=64)

    out = _global(x2, q2x[:, :, None], q2y[:, :, None],
                  W3a[:256], W3a[256:257], W3a[257:258], b3a[None],
                  W3b, b3b[None])
    return out.reshape(_B, -1)


# speedup vs baseline: 10.6775x; 10.6775x over previous
import functools

import numpy as np
import jax
import jax.numpy as jnp
from jax.experimental import pallas as pl
from jax.experimental.pallas import tpu as pltpu

_B = 4
_N1 = 1024
_M1 = 256
_M2 = 64
_R2_1 = 0.4 * 0.4
_R2_2 = 0.8 * 0.8


def _fps_kernel(px_ref, py_ref, q1x_ref, q1y_ref, q2x_ref, q2y_ref):
    def run(px, py, m, qx_ref, qy_ref):
        b, n = px.shape
        iota = jax.lax.broadcasted_iota(jnp.int32, (1, n), 1)
        miota = jax.lax.broadcasted_iota(jnp.int32, (1, m), 1)

        def body(i, carry):
            dist, last, qx, qy = carry
            sel = (iota == last).astype(jnp.float32)
            lx = jnp.sum(px * sel, axis=1, keepdims=True)
            ly = jnp.sum(py * sel, axis=1, keepdims=True)
            at = miota == (i - 1)
            qx = jnp.where(at, lx, qx)
            qy = jnp.where(at, ly, qy)
            d = (px - lx) ** 2 + (py - ly) ** 2
            dist = jnp.minimum(dist, d)
            mx = jnp.max(dist, axis=1, keepdims=True)
            nxt = jnp.min(jnp.where(dist == mx, iota, n), axis=1, keepdims=True)
            return dist, nxt, qx, qy

        dist0 = jnp.full((b, n), jnp.inf, dtype=jnp.float32)
        last0 = jnp.zeros((b, 1), dtype=jnp.int32)
        qx0 = jnp.zeros((b, m), dtype=jnp.float32)
        _, _, qx, qy = jax.lax.fori_loop(1, m + 1, body,
                                         (dist0, last0, qx0, qx0))
        qx_ref[...] = qx
        qy_ref[...] = qy

    run(px_ref[...], py_ref[...], _M1, q1x_ref, q1y_ref)
    run(q1x_ref[...], q1y_ref[...], _M2, q2x_ref, q2y_ref)


def _fps(px, py):
    f32 = jnp.float32
    return pl.pallas_call(
        _fps_kernel,
        out_shape=[
            jax.ShapeDtypeStruct((_B, _M1), f32),
            jax.ShapeDtypeStruct((_B, _M1), f32),
            jax.ShapeDtypeStruct((_B, _M2), f32),
            jax.ShapeDtypeStruct((_B, _M2), f32),
        ],
    )(px, py)


def _atab_kernel(x_ref, sx_ref, sy_ref, w_ref, b_ref, wpx_ref, wpy_ref, o_ref):
    x = x_ref[...].reshape(x_ref.shape[1:])
    sx = sx_ref[...].reshape(sx_ref.shape[1:])
    sy = sy_ref[...].reshape(sy_ref.shape[1:])
    a = jnp.dot(x, w_ref[...], preferred_element_type=jnp.float32)
    a = a + b_ref[...] + sx * wpx_ref[...] + sy * wpy_ref[...]
    o_ref[...] = a.reshape(o_ref.shape)


def _atab(x, sx3, sy3, w, b, wpx, wpy):
    bsz, n, c = x.shape
    h = w.shape[1]
    return pl.pallas_call(
        _atab_kernel,
        grid=(bsz,),
        in_specs=[
            pl.BlockSpec((1, n, c), lambda i: (i, 0, 0)),
            pl.BlockSpec((1, n, 1), lambda i: (i, 0, 0)),
            pl.BlockSpec((1, n, 1), lambda i: (i, 0, 0)),
            pl.BlockSpec((c, h), lambda i: (0, 0)),
            pl.BlockSpec((1, h), lambda i: (0, 0)),
            pl.BlockSpec((1, h), lambda i: (0, 0)),
            pl.BlockSpec((1, h), lambda i: (0, 0)),
        ],
        out_specs=pl.BlockSpec((1, n, h), lambda i: (i, 0, 0)),
        out_shape=jax.ShapeDtypeStruct((bsz, n, h), jnp.float32),
    )(x, sx3, sy3, w, b, wpx, wpy)


def _bq_kernel(qx_ref, qy_ref, wpx_ref, wpy_ref, o_ref):
    qx = qx_ref[...].reshape(qx_ref.shape[1:])
    qy = qy_ref[...].reshape(qy_ref.shape[1:])
    o = -(qx * wpx_ref[...] + qy * wpy_ref[...])
    o_ref[...] = o.reshape(o_ref.shape)


def _bq(qx3, qy3, wpx, wpy):
    bsz, m, _ = qx3.shape
    h = wpx.shape[1]
    return pl.pallas_call(
        _bq_kernel,
        grid=(bsz,),
        in_specs=[
            pl.BlockSpec((1, m, 1), lambda i: (i, 0, 0)),
            pl.BlockSpec((1, m, 1), lambda i: (i, 0, 0)),
            pl.BlockSpec((1, h), lambda i: (0, 0)),
            pl.BlockSpec((1, h), lambda i: (0, 0)),
        ],
        out_specs=pl.BlockSpec((1, m, h), lambda i: (i, 0, 0)),
        out_shape=jax.ShapeDtypeStruct((bsz, m, h), jnp.float32),
    )(qx3, qy3, wpx, wpy)


def _sa_kernel(a_ref, sx_ref, sy_ref, qx_ref, qy_ref, bq_ref,
               wb_ref, bb_ref, o_ref, acc_ref, *, r2, ns):
    s = pl.program_id(2)

    @pl.when(s == 0)
    def _():
        acc_ref[...] = jnp.full(acc_ref.shape, -jnp.inf, dtype=jnp.float32)

    a = a_ref[...].reshape(a_ref.shape[1:])
    sx = sx_ref[...].reshape(sx_ref.shape[2:])
    sy = sy_ref[...].reshape(sy_ref.shape[2:])
    qx = qx_ref[...].reshape(qx_ref.shape[1:])
    qy = qy_ref[...].reshape(qy_ref.shape[1:])
    ssz, h = a.shape
    tq = qx.shape[0]
    bq = bq_ref[...].reshape(tq, 1, h)
    pre = (jnp.broadcast_to(a[None, :, :], (tq, ssz, h))
           + jnp.broadcast_to(bq, (tq, ssz, h)))
    t = jnp.tanh(pre).reshape(tq * ssz, h)
    ho = wb_ref.shape[1]
    hh = jnp.dot(t, wb_ref[...], preferred_element_type=jnp.float32)
    hh = (hh + bb_ref[...]).reshape(tq, ssz, ho)
    relx = sx - qx
    rely = sy - qy
    d2 = relx * relx + rely * rely
    pen = jnp.where(d2 <= r2, 0.0, -jnp.inf).astype(jnp.float32)
    hm = hh + pen[:, :, None]
    acc_ref[...] = jnp.maximum(acc_ref[...], jnp.max(hm, axis=1))

    @pl.when(s == ns - 1)
    def _():
        o_ref[...] = acc_ref[...].reshape(o_ref.shape)


def _sa(a, sx, sy, qx3, qy3, bq, wb, bb, r2, tq, schunk):
    bsz, n, h = a.shape
    m = qx3.shape[1]
    ho = wb.shape[1]
    nq = m // tq
    ns = n // schunk
    sx4 = sx.reshape(bsz, ns, 1, schunk)
    sy4 = sy.reshape(bsz, ns, 1, schunk)
    bq5 = bq.reshape(bsz, nq, tq, 1, h)
    kern = functools.partial(_sa_kernel, r2=r2, ns=ns)
    return pl.pallas_call(
        kern,
        grid=(bsz, nq, ns),
        in_specs=[
            pl.BlockSpec((1, schunk, h), lambda b, q, s: (b, s, 0)),
            pl.BlockSpec((1, 1, 1, schunk), lambda b, q, s: (b, s, 0, 0)),
            pl.BlockSpec((1, 1, 1, schunk), lambda b, q, s: (b, s, 0, 0)),
            pl.BlockSpec((1, tq, 1), lambda b, q, s: (b, q, 0)),
            pl.BlockSpec((1, tq, 1), lambda b, q, s: (b, q, 0)),
            pl.BlockSpec((1, 1, tq, 1, h), lambda b, q, s: (b, q, 0, 0, 0)),
            pl.BlockSpec((h, ho), lambda b, q, s: (0, 0)),
            pl.BlockSpec((1, ho), lambda b, q, s: (0, 0)),
        ],
        out_specs=pl.BlockSpec((1, tq, ho), lambda b, q, s: (b, q, 0)),
        out_shape=jax.ShapeDtypeStruct((bsz, m, ho), jnp.float32),
        scratch_shapes=[pltpu.VMEM((tq, ho), jnp.float32)],
        compiler_params=pltpu.CompilerParams(
            dimension_semantics=("parallel", "parallel", "arbitrary"),
        ),
    )(a, sx4, sy4, qx3, qy3, bq5, wb, bb)


def _global_kernel(x2_ref, qx_ref, qy_ref, w3x_ref, w3px_ref, w3py_ref,
                   b3a_ref, w3b_ref, b3b_ref, o_ref):
    x2 = x2_ref[...].reshape(x2_ref.shape[1:])
    qx = qx_ref[...].reshape(qx_ref.shape[1:])
    qy = qy_ref[...].reshape(qy_ref.shape[1:])
    pre = jnp.dot(x2, w3x_ref[...], preferred_element_type=jnp.float32)
    pre = pre + qx * w3px_ref[...] + qy * w3py_ref[...] + b3a_ref[...]
    hh = jnp.dot(jnp.tanh(pre), w3b_ref[...],
                 preferred_element_type=jnp.float32) + b3b_ref[...]
    o_ref[...] = jnp.max(hh, axis=0, keepdims=True)[None]


def _global(x2, qx3, qy3, w3x, w3px, w3py, b3a, w3b, b3b):
    bsz, m, c = x2.shape
    h1 = w3x.shape[1]
    h2 = w3b.shape[1]
    return pl.pallas_call(
        _global_kernel,
        grid=(bsz,),
        in_specs=[
            pl.BlockSpec((1, m, c), lambda i: (i, 0, 0)),
            pl.BlockSpec((1, m, 1), lambda i: (i, 0, 0)),
            pl.BlockSpec((1, m, 1), lambda i: (i, 0, 0)),
            pl.BlockSpec((c, h1), lambda i: (0, 0)),
            pl.BlockSpec((1, h1), lambda i: (0, 0)),
            pl.BlockSpec((1, h1), lambda i: (0, 0)),
            pl.BlockSpec((1, h1), lambda i: (0, 0)),
            pl.BlockSpec((h1, h2), lambda i: (0, 0)),
            pl.BlockSpec((1, h2), lambda i: (0, 0)),
        ],
        out_specs=pl.BlockSpec((1, 1, h2), lambda i: (i, 0, 0)),
        out_shape=jax.ShapeDtypeStruct((bsz, 1, h2), jnp.float32),
    )(x2, qx3, qy3, w3x, w3px, w3py, b3a, w3b, b3b)


def kernel(x, pos, W1a, b1a, W1b, b1b, W2a, b2a, W2b, b2b, W3a, b3a, W3b, b3b):
    px = pos[:, :, 0]
    py = pos[:, :, 1]
    q1x, q1y, q2x, q2y = _fps(px, py)

    a1 = _atab(x, px[:, :, None], py[:, :, None],
               W1a[:64], b1a[None], W1a[64:65], W1a[65:66])
    bq1 = _bq(q1x[:, :, None], q1y[:, :, None], W1a[64:65], W1a[65:66])
    x1 = _sa(a1, px, py, q1x[:, :, None], q1y[:, :, None], bq1,
             W1b, b1b[None], r2=_R2_1, tq=64, schunk=128)

    a2 = _atab(x1, q1x[:, :, None], q1y[:, :, None],
               W2a[:128], b2a[None], W2a[128:129], W2a[129:130])
    bq2 = _bq(q2x[:, :, None], q2y[:, :, None], W2a[128:129], W2a[129:130])
    x2 = _sa(a2, q1x, q1y, q2x[:, :, None], q2y[:, :, None], bq2,
             W2b, b2b[None], r2=_R2_2, tq=64, schunk=64)

    out = _global(x2, q2x[:, :, None], q2y[:, :, None],
                  W3a[:256], W3a[256:257], W3a[257:258], b3a[None],
                  W3b, b3b[None])
    return out.reshape(_B, -1)


# speedup vs baseline: 11.3001x; 1.0583x over previous
import functools

import numpy as np
import jax
import jax.numpy as jnp
from jax.experimental import pallas as pl
from jax.experimental.pallas import tpu as pltpu

_B = 4
_N1 = 1024
_M1 = 256
_M2 = 64
_R2_1 = 0.4 * 0.4
_R2_2 = 0.8 * 0.8


def _fps_kernel(px_ref, py_ref, q1x_ref, q1y_ref, q2x_ref, q2y_ref):
    def run(px, py, m, qx_ref, qy_ref):
        b, n = px.shape
        iota = jax.lax.broadcasted_iota(jnp.int32, (1, n), 1)
        miota = jax.lax.broadcasted_iota(jnp.int32, (1, m), 1)

        def body(i, carry):
            dist, last, qx, qy = carry
            sel = (iota == last).astype(jnp.float32)
            lx = jnp.sum(px * sel, axis=1, keepdims=True)
            ly = jnp.sum(py * sel, axis=1, keepdims=True)
            at = miota == (i - 1)
            qx = jnp.where(at, lx, qx)
            qy = jnp.where(at, ly, qy)
            d = (px - lx) ** 2 + (py - ly) ** 2
            dist = jnp.minimum(dist, d)
            mx = jnp.max(dist, axis=1, keepdims=True)
            nxt = jnp.min(jnp.where(dist == mx, iota, n), axis=1, keepdims=True)
            return dist, nxt, qx, qy

        dist0 = jnp.full((b, n), jnp.inf, dtype=jnp.float32)
        last0 = jnp.zeros((b, 1), dtype=jnp.int32)
        qx0 = jnp.zeros((b, m), dtype=jnp.float32)
        _, _, qx, qy = jax.lax.fori_loop(1, m + 1, body,
                                         (dist0, last0, qx0, qx0))
        qx_ref[...] = qx
        qy_ref[...] = qy

    run(px_ref[...], py_ref[...], _M1, q1x_ref, q1y_ref)
    run(q1x_ref[...], q1y_ref[...], _M2, q2x_ref, q2y_ref)


def _fps(px, py):
    f32 = jnp.float32
    return pl.pallas_call(
        _fps_kernel,
        out_shape=[
            jax.ShapeDtypeStruct((_B, _M1), f32),
            jax.ShapeDtypeStruct((_B, _M1), f32),
            jax.ShapeDtypeStruct((_B, _M2), f32),
            jax.ShapeDtypeStruct((_B, _M2), f32),
        ],
    )(px, py)


def _atab_kernel(x_ref, sx_ref, sy_ref, w_ref, b_ref, wpx_ref, wpy_ref, o_ref):
    x = x_ref[...].reshape(x_ref.shape[1:])
    sx = sx_ref[...].reshape(sx_ref.shape[1:])
    sy = sy_ref[...].reshape(sy_ref.shape[1:])
    a = jnp.dot(x, w_ref[...], preferred_element_type=jnp.float32)
    a = a + b_ref[...] + sx * wpx_ref[...] + sy * wpy_ref[...]
    o_ref[...] = a.reshape(o_ref.shape)


def _atab(x, sx3, sy3, w, b, wpx, wpy):
    bsz, n, c = x.shape
    h = w.shape[1]
    return pl.pallas_call(
        _atab_kernel,
        grid=(bsz,),
        in_specs=[
            pl.BlockSpec((1, n, c), lambda i: (i, 0, 0)),
            pl.BlockSpec((1, n, 1), lambda i: (i, 0, 0)),
            pl.BlockSpec((1, n, 1), lambda i: (i, 0, 0)),
            pl.BlockSpec((c, h), lambda i: (0, 0)),
            pl.BlockSpec((1, h), lambda i: (0, 0)),
            pl.BlockSpec((1, h), lambda i: (0, 0)),
            pl.BlockSpec((1, h), lambda i: (0, 0)),
        ],
        out_specs=pl.BlockSpec((1, n, h), lambda i: (i, 0, 0)),
        out_shape=jax.ShapeDtypeStruct((bsz, n, h), jnp.float32),
    )(x, sx3, sy3, w, b, wpx, wpy)


def _bq_kernel(qx_ref, qy_ref, wpx_ref, wpy_ref, o_ref):
    qx = qx_ref[...].reshape(qx_ref.shape[1:])
    qy = qy_ref[...].reshape(qy_ref.shape[1:])
    o = -(qx * wpx_ref[...] + qy * wpy_ref[...])
    o_ref[...] = o.reshape(o_ref.shape)


def _bq(qx3, qy3, wpx, wpy):
    bsz, m, _ = qx3.shape
    h = wpx.shape[1]
    return pl.pallas_call(
        _bq_kernel,
        grid=(bsz,),
        in_specs=[
            pl.BlockSpec((1, m, 1), lambda i: (i, 0, 0)),
            pl.BlockSpec((1, m, 1), lambda i: (i, 0, 0)),
            pl.BlockSpec((1, h), lambda i: (0, 0)),
            pl.BlockSpec((1, h), lambda i: (0, 0)),
        ],
        out_specs=pl.BlockSpec((1, m, h), lambda i: (i, 0, 0)),
        out_shape=jax.ShapeDtypeStruct((bsz, m, h), jnp.float32),
    )(qx3, qy3, wpx, wpy)


def _sa_kernel(a_ref, sx_ref, sy_ref, qx_ref, qy_ref, bq_ref,
               wb_ref, bb_ref, o_ref, acc_ref, *, r2, ns):
    s = pl.program_id(2)

    @pl.when(s == 0)
    def _():
        acc_ref[...] = jnp.full(acc_ref.shape, -jnp.inf, dtype=jnp.float32)

    a = a_ref[...].reshape(a_ref.shape[1:])
    sx = sx_ref[...].reshape(sx_ref.shape[2:])
    sy = sy_ref[...].reshape(sy_ref.shape[2:])
    qx = qx_ref[...].reshape(qx_ref.shape[1:])
    qy = qy_ref[...].reshape(qy_ref.shape[1:])
    ssz, h = a.shape
    tq = qx.shape[0]
    bq = bq_ref[...].reshape(tq, 1, h)
    pre = (jnp.broadcast_to(a[None, :, :], (tq, ssz, h))
           + jnp.broadcast_to(bq, (tq, ssz, h)))
    t = jnp.tanh(pre).reshape(tq * ssz, h)
    ho = wb_ref.shape[1]
    hh = jnp.dot(t, wb_ref[...], preferred_element_type=jnp.float32)
    hh = (hh + bb_ref[...]).reshape(tq, ssz, ho)
    relx = sx - qx
    rely = sy - qy
    d2 = relx * relx + rely * rely
    pen = jnp.where(d2 <= r2, 0.0, -jnp.inf).astype(jnp.float32)
    hm = hh + pen[:, :, None]
    acc_ref[...] = jnp.maximum(acc_ref[...], jnp.max(hm, axis=1))

    @pl.when(s == ns - 1)
    def _():
        o_ref[...] = acc_ref[...].reshape(o_ref.shape)


def _sa(a, sx, sy, qx3, qy3, bq, wb, bb, r2, tq, schunk):
    bsz, n, h = a.shape
    m = qx3.shape[1]
    ho = wb.shape[1]
    nq = m // tq
    ns = n // schunk
    sx4 = sx.reshape(bsz, ns, 1, schunk)
    sy4 = sy.reshape(bsz, ns, 1, schunk)
    bq5 = bq.reshape(bsz, nq, tq, 1, h)
    kern = functools.partial(_sa_kernel, r2=r2, ns=ns)
    return pl.pallas_call(
        kern,
        grid=(bsz, nq, ns),
        in_specs=[
            pl.BlockSpec((1, schunk, h), lambda b, q, s: (b, s, 0)),
            pl.BlockSpec((1, 1, 1, schunk), lambda b, q, s: (b, s, 0, 0)),
            pl.BlockSpec((1, 1, 1, schunk), lambda b, q, s: (b, s, 0, 0)),
            pl.BlockSpec((1, tq, 1), lambda b, q, s: (b, q, 0)),
            pl.BlockSpec((1, tq, 1), lambda b, q, s: (b, q, 0)),
            pl.BlockSpec((1, 1, tq, 1, h), lambda b, q, s: (b, q, 0, 0, 0)),
            pl.BlockSpec((h, ho), lambda b, q, s: (0, 0)),
            pl.BlockSpec((1, ho), lambda b, q, s: (0, 0)),
        ],
        out_specs=pl.BlockSpec((1, tq, ho), lambda b, q, s: (b, q, 0)),
        out_shape=jax.ShapeDtypeStruct((bsz, m, ho), jnp.float32),
        scratch_shapes=[pltpu.VMEM((tq, ho), jnp.float32)],
        compiler_params=pltpu.CompilerParams(
            dimension_semantics=("parallel", "parallel", "arbitrary"),
        ),
    )(a, sx4, sy4, qx3, qy3, bq5, wb, bb)


def _global_kernel(x2_ref, qx_ref, qy_ref, w3x_ref, w3px_ref, w3py_ref,
                   b3a_ref, w3b_ref, b3b_ref, o_ref):
    x2 = x2_ref[...].reshape(x2_ref.shape[1:])
    qx = qx_ref[...].reshape(qx_ref.shape[1:])
    qy = qy_ref[...].reshape(qy_ref.shape[1:])
    pre = jnp.dot(x2, w3x_ref[...], preferred_element_type=jnp.float32)
    pre = pre + qx * w3px_ref[...] + qy * w3py_ref[...] + b3a_ref[...]
    hh = jnp.dot(jnp.tanh(pre), w3b_ref[...],
                 preferred_element_type=jnp.float32) + b3b_ref[...]
    o_ref[...] = jnp.max(hh, axis=0, keepdims=True)[None]


def _global(x2, qx3, qy3, w3x, w3px, w3py, b3a, w3b, b3b):
    bsz, m, c = x2.shape
    h1 = w3x.shape[1]
    h2 = w3b.shape[1]
    return pl.pallas_call(
        _global_kernel,
        grid=(bsz,),
        in_specs=[
            pl.BlockSpec((1, m, c), lambda i: (i, 0, 0)),
            pl.BlockSpec((1, m, 1), lambda i: (i, 0, 0)),
            pl.BlockSpec((1, m, 1), lambda i: (i, 0, 0)),
            pl.BlockSpec((c, h1), lambda i: (0, 0)),
            pl.BlockSpec((1, h1), lambda i: (0, 0)),
            pl.BlockSpec((1, h1), lambda i: (0, 0)),
            pl.BlockSpec((1, h1), lambda i: (0, 0)),
            pl.BlockSpec((h1, h2), lambda i: (0, 0)),
            pl.BlockSpec((1, h2), lambda i: (0, 0)),
        ],
        out_specs=pl.BlockSpec((1, 1, h2), lambda i: (i, 0, 0)),
        out_shape=jax.ShapeDtypeStruct((bsz, 1, h2), jnp.float32),
    )(x2, qx3, qy3, w3x, w3px, w3py, b3a, w3b, b3b)


def kernel(x, pos, W1a, b1a, W1b, b1b, W2a, b2a, W2b, b2b, W3a, b3a, W3b, b3b):
    px = pos[:, :, 0]
    py = pos[:, :, 1]
    q1x, q1y, q2x, q2y = _fps(px, py)

    a1 = _atab(x, px[:, :, None], py[:, :, None],
               W1a[:64], b1a[None], W1a[64:65], W1a[65:66])
    bq1 = _bq(q1x[:, :, None], q1y[:, :, None], W1a[64:65], W1a[65:66])
    x1 = _sa(a1, px, py, q1x[:, :, None], q1y[:, :, None], bq1,
             W1b, b1b[None], r2=_R2_1, tq=64, schunk=256)

    a2 = _atab(x1, q1x[:, :, None], q1y[:, :, None],
               W2a[:128], b2a[None], W2a[128:129], W2a[129:130])
    bq2 = _bq(q2x[:, :, None], q2y[:, :, None], W2a[128:129], W2a[129:130])
    x2 = _sa(a2, q1x, q1y, q2x[:, :, None], q2y[:, :, None], bq2,
             W2b, b2b[None], r2=_R2_2, tq=64, schunk=64)

    out = _global(x2, q2x[:, :, None], q2y[:, :, None],
                  W3a[:256], W3a[256:257], W3a[257:258], b3a[None],
                  W3b, b3b[None])
    return out.reshape(_B, -1)
